# baseline (speedup 1.0000x reference)
import functools

import jax
import jax.numpy as jnp
from jax import lax
from jax.experimental import pallas as pl
from jax.experimental.pallas import tpu as pltpu
from jax.experimental.pallas import tpu_sc as plsc

RANK = 64
D_MODEL = 768

NUM_CORES = 2
NUM_SUBCORES = 16
NW = NUM_CORES * NUM_SUBCORES

CHUNK = 128


def _sc_gather(ids_flat, table):
    n = ids_flat.shape[0]
    assert n % (NW * CHUNK) == 0
    b_per_w = n // NW
    n_chunks = b_per_w // CHUNK
    idx3 = ids_flat.reshape(NW, n_chunks, CHUNK)

    mesh = plsc.VectorSubcoreMesh(
        core_axis_name="c", subcore_axis_name="s",
        num_cores=NUM_CORES, num_subcores=NUM_SUBCORES)

    @functools.partial(
        pl.kernel,
        out_type=jax.ShapeDtypeStruct((n, RANK), jnp.float32),
        mesh=mesh,
        scratch_types=[
            pltpu.VMEM((n_chunks, CHUNK), jnp.int32),
            pltpu.VMEM((CHUNK, RANK), jnp.float32),
            pltpu.SemaphoreType.DMA,
        ],
        compiler_params=pltpu.CompilerParams(use_tc_tiling_on_sc=False),
    )
    def gather_kernel(idx_hbm, table_hbm, out_hbm, idx_v, rows_v, sem):
        wid = lax.axis_index("s") * NUM_CORES + lax.axis_index("c")
        base = wid * b_per_w
        pltpu.sync_copy(idx_hbm.at[wid], idx_v)

        def body(i, carry):
            pltpu.async_copy(table_hbm.at[idx_v.at[i]], rows_v, sem).wait()
            pltpu.sync_copy(rows_v, out_hbm.at[pl.ds(base + i * CHUNK, CHUNK)])
            return carry

        lax.fori_loop(0, n_chunks, body, 0)

    return gather_kernel(idx3, table)


def _tc_project(low_rank, proj_w):
    n = low_rank.shape[0]
    bm = 1024
    assert n % bm == 0

    def mm_kernel(x_ref, w_ref, o_ref):
        o_ref[...] = jnp.dot(x_ref[...], w_ref[...],
                             preferred_element_type=jnp.float32)

    return pl.pallas_call(
        mm_kernel,
        grid=(n // bm,),
        in_specs=[
            pl.BlockSpec((bm, RANK), lambda i: (i, 0)),
            pl.BlockSpec((RANK, D_MODEL), lambda i: (0, 0)),
        ],
        out_specs=pl.BlockSpec((bm, D_MODEL), lambda i: (i, 0)),
        out_shape=jax.ShapeDtypeStruct((n, D_MODEL), jnp.float32),
    )(low_rank, proj_w)


def kernel(input_ids, embed_table, proj_w):
    b, l = input_ids.shape
    ids_flat = input_ids.reshape(-1).astype(jnp.int32)
    low_rank = _sc_gather(ids_flat, embed_table)
    out = _tc_project(low_rank, proj_w)
    return out.reshape(b, l, D_MODEL)


# speedup vs baseline: 1.3108x; 1.3108x over previous
import functools

import jax
import jax.numpy as jnp
from jax import lax
from jax.experimental import pallas as pl
from jax.experimental.pallas import tpu as pltpu
from jax.experimental.pallas import tpu_sc as plsc

RANK = 64
D_MODEL = 768

NUM_CORES = 2
NUM_SUBCORES = 16
NW = NUM_CORES * NUM_SUBCORES

CHUNK = 128


def _sc_gather(idx3, table):
    nw, n_chunks, chunk = idx3.shape
    n = nw * n_chunks * chunk
    b_per_w = n_chunks * chunk

    mesh = plsc.VectorSubcoreMesh(
        core_axis_name="c", subcore_axis_name="s",
        num_cores=NUM_CORES, num_subcores=NUM_SUBCORES)

    @functools.partial(
        pl.kernel,
        out_type=jax.ShapeDtypeStruct((n, RANK), jnp.float32),
        mesh=mesh,
        scratch_types=[
            pltpu.VMEM((n_chunks, chunk), jnp.int32),
            pltpu.VMEM((chunk, RANK), jnp.float32),
            pltpu.SemaphoreType.DMA,
        ],
        compiler_params=pltpu.CompilerParams(use_tc_tiling_on_sc=False),
    )
    def gather_kernel(idx_hbm, table_hbm, out_hbm, idx_v, rows_v, sem):
        wid = lax.axis_index("s") * NUM_CORES + lax.axis_index("c")
        base = wid * b_per_w
        pltpu.sync_copy(idx_hbm.at[wid], idx_v)

        def body(i, carry):
            pltpu.async_copy(table_hbm.at[idx_v.at[i]], rows_v, sem).wait()
            pltpu.sync_copy(rows_v, out_hbm.at[pl.ds(base + i * chunk, chunk)])
            return carry

        lax.fori_loop(0, n_chunks, body, 0)

    return gather_kernel(idx3, table)


def _tc_project(low2, proj_w2):
    n2 = low2.shape[0]
    bm = 512
    assert n2 % bm == 0

    def mm_kernel(x_ref, w_ref, o_ref):
        o_ref[...] = jnp.dot(x_ref[...], w_ref[...],
                             preferred_element_type=jnp.float32)

    return pl.pallas_call(
        mm_kernel,
        grid=(n2 // bm,),
        in_specs=[
            pl.BlockSpec((bm, 2 * RANK), lambda i: (i, 0)),
            pl.BlockSpec((2 * RANK, 2 * D_MODEL), lambda i: (0, 0)),
        ],
        out_specs=pl.BlockSpec((bm, 2 * D_MODEL), lambda i: (i, 0)),
        out_shape=jax.ShapeDtypeStruct((n2, 2 * D_MODEL), jnp.float32),
    )(low2, proj_w2)


def kernel(input_ids, embed_table, proj_w):
    b, l = input_ids.shape
    n = b * l
    ids_lmajor = input_ids.T.astype(jnp.int32).reshape(NW, n // (NW * CHUNK),
                                                       CHUNK)
    low_rank = _sc_gather(ids_lmajor, embed_table)
    low2 = low_rank.reshape(n // 2, 2 * RANK)
    w2 = jnp.zeros((2 * RANK, 2 * D_MODEL), proj_w.dtype)
    w2 = w2.at[:RANK, :D_MODEL].set(proj_w).at[RANK:, D_MODEL:].set(proj_w)
    out2 = _tc_project(low2, w2)
    return out2.reshape(l, b, D_MODEL).transpose(1, 0, 2)


# speedup vs baseline: 1.7850x; 1.3618x over previous
import functools

import jax
import jax.numpy as jnp
from jax import lax
from jax.experimental import pallas as pl
from jax.experimental.pallas import tpu as pltpu
from jax.experimental.pallas import tpu_sc as plsc

RANK = 64
D_MODEL = 768

NUM_CORES = 2
NUM_SUBCORES = 16
NW = NUM_CORES * NUM_SUBCORES

CHUNK = 128


def _sc_gather(idx3, table):
    nw, n_chunks, chunk = idx3.shape
    n = nw * n_chunks * chunk
    b_per_w = n_chunks * chunk

    mesh = plsc.VectorSubcoreMesh(
        core_axis_name="c", subcore_axis_name="s",
        num_cores=NUM_CORES, num_subcores=NUM_SUBCORES)

    @functools.partial(
        pl.kernel,
        out_type=jax.ShapeDtypeStruct((n, RANK), jnp.float32),
        mesh=mesh,
        scratch_types=[
            pltpu.VMEM((n_chunks, chunk), jnp.int32),
            pltpu.VMEM((chunk, RANK), jnp.float32),
            pltpu.SemaphoreType.DMA,
        ],
        compiler_params=pltpu.CompilerParams(use_tc_tiling_on_sc=False),
    )
    def gather_kernel(idx_hbm, table_hbm, out_hbm, idx_v, rows_v, sem):
        wid = lax.axis_index("s") * NUM_CORES + lax.axis_index("c")
        base = wid * b_per_w
        pltpu.sync_copy(idx_hbm.at[wid], idx_v)

        def body(i, carry):
            pltpu.async_copy(table_hbm.at[idx_v.at[i]], rows_v, sem).wait()
            pltpu.sync_copy(rows_v, out_hbm.at[pl.ds(base + i * chunk, chunk)])
            return carry

        lax.fori_loop(0, n_chunks, body, 0)

    return gather_kernel(idx3, table)


def _tc_project(low2, proj_w2, b, l):
    n2 = low2.shape[0]
    bm = 512
    assert b % bm == 0

    def mm_kernel(x_ref, w_ref, o_ref):
        y = jnp.dot(x_ref[...], w_ref[...],
                    preferred_element_type=jnp.float32)
        o_ref[0] = y.reshape(bm, D_MODEL)

    grid = (l, b // bm)
    return pl.pallas_call(
        mm_kernel,
        grid=grid,
        in_specs=[
            pl.BlockSpec((bm // 2, 2 * RANK),
                         lambda i, j: (i * (b // bm) + j, 0)),
            pl.BlockSpec((2 * RANK, 2 * D_MODEL), lambda i, j: (0, 0)),
        ],
        out_specs=pl.BlockSpec((1, bm, D_MODEL), lambda i, j: (i, j, 0)),
        out_shape=jax.ShapeDtypeStruct((l, b, D_MODEL), jnp.float32),
    )(low2, proj_w2)


def kernel(input_ids, embed_table, proj_w):
    b, l = input_ids.shape
    n = b * l
    ids_lmajor = input_ids.T.astype(jnp.int32).reshape(NW, n // (NW * CHUNK),
                                                       CHUNK)
    low_rank = _sc_gather(ids_lmajor, embed_table)
    low2 = low_rank.reshape(n // 2, 2 * RANK)
    w2 = jnp.zeros((2 * RANK, 2 * D_MODEL), proj_w.dtype)
    w2 = w2.at[:RANK, :D_MODEL].set(proj_w).at[RANK:, D_MODEL:].set(proj_w)
    out3 = _tc_project(low2, w2, b, l)
    return out3.transpose(1, 0, 2)


# speedup vs baseline: 1.8961x; 1.0622x over previous
import functools

import jax
import jax.numpy as jnp
from jax import lax
from jax.experimental import pallas as pl
from jax.experimental.pallas import tpu as pltpu
from jax.experimental.pallas import tpu_sc as plsc

RANK = 64
D_MODEL = 768

NUM_CORES = 2
NUM_SUBCORES = 16
NW = NUM_CORES * NUM_SUBCORES

CHUNK = 128


def _sc_gather(idx3, table):
    nw, n_chunks, chunk = idx3.shape
    n = nw * n_chunks * chunk
    b_per_w = n_chunks * chunk

    mesh = plsc.VectorSubcoreMesh(
        core_axis_name="c", subcore_axis_name="s",
        num_cores=NUM_CORES, num_subcores=NUM_SUBCORES)

    @functools.partial(
        pl.kernel,
        out_type=jax.ShapeDtypeStruct((n, RANK), jnp.float32),
        mesh=mesh,
        scratch_types=[
            pltpu.VMEM((n_chunks, chunk), jnp.int32),
            pltpu.VMEM((chunk, RANK), jnp.float32),
            pltpu.SemaphoreType.DMA,
        ],
        compiler_params=pltpu.CompilerParams(use_tc_tiling_on_sc=False),
    )
    def gather_kernel(idx_hbm, table_hbm, out_hbm, idx_v, rows_v, sem):
        wid = lax.axis_index("s") * NUM_CORES + lax.axis_index("c")
        base = wid * b_per_w
        pltpu.sync_copy(idx_hbm.at[wid], idx_v)

        def body(i, carry):
            pltpu.async_copy(table_hbm.at[idx_v.at[i]], rows_v, sem).wait()
            pltpu.sync_copy(rows_v, out_hbm.at[pl.ds(base + i * chunk, chunk)])
            return carry

        lax.fori_loop(0, n_chunks, body, 0)

    return gather_kernel(idx3, table)


def _tc_project(low2, proj_w2, b, l):
    n2 = low2.shape[0]
    bm = 1024
    assert b % bm == 0

    def mm_kernel(x_ref, w_ref, o_ref):
        y = jnp.dot(x_ref[...], w_ref[...],
                    preferred_element_type=jnp.float32)
        o_ref[0] = y.reshape(bm, D_MODEL)

    grid = (l, b // bm)
    return pl.pallas_call(
        mm_kernel,
        grid=grid,
        in_specs=[
            pl.BlockSpec((bm // 2, 2 * RANK),
                         lambda i, j: (i * (b // bm) + j, 0)),
            pl.BlockSpec((2 * RANK, 2 * D_MODEL), lambda i, j: (0, 0)),
        ],
        out_specs=pl.BlockSpec((1, bm, D_MODEL), lambda i, j: (i, j, 0)),
        out_shape=jax.ShapeDtypeStruct((l, b, D_MODEL), jnp.float32),
    )(low2, proj_w2)


def kernel(input_ids, embed_table, proj_w):
    b, l = input_ids.shape
    n = b * l
    ids_lmajor = input_ids.T.astype(jnp.int32).reshape(NW, n // (NW * CHUNK),
                                                       CHUNK)
    low_rank = _sc_gather(ids_lmajor, embed_table)
    low2 = low_rank.reshape(n // 2, 2 * RANK)
    w2 = jnp.zeros((2 * RANK, 2 * D_MODEL), proj_w.dtype)
    w2 = w2.at[:RANK, :D_MODEL].set(proj_w).at[RANK:, D_MODEL:].set(proj_w)
    out3 = _tc_project(low2, w2, b, l)
    return out3.transpose(1, 0, 2)


# speedup vs baseline: 2.0883x; 1.1013x over previous
import functools

import jax
import jax.numpy as jnp
from jax import lax
from jax.experimental import pallas as pl
from jax.experimental.pallas import tpu as pltpu
from jax.experimental.pallas import tpu_sc as plsc

RANK = 64
D_MODEL = 768

NUM_CORES = 2
NUM_SUBCORES = 16
NW = NUM_CORES * NUM_SUBCORES

CHUNK = 128


def _sc_gather(idx3, table):
    nw, n_chunks, chunk = idx3.shape
    n = nw * n_chunks * chunk
    b_per_w = n_chunks * chunk

    mesh = plsc.VectorSubcoreMesh(
        core_axis_name="c", subcore_axis_name="s",
        num_cores=NUM_CORES, num_subcores=NUM_SUBCORES)

    @functools.partial(
        pl.kernel,
        out_type=jax.ShapeDtypeStruct((n, RANK), jnp.float32),
        mesh=mesh,
        scratch_types=[
            pltpu.VMEM((n_chunks, chunk), jnp.int32),
            pltpu.VMEM((chunk, RANK), jnp.float32),
            pltpu.SemaphoreType.DMA,
        ],
        compiler_params=pltpu.CompilerParams(use_tc_tiling_on_sc=False),
    )
    def gather_kernel(idx_hbm, table_hbm, out_hbm, idx_v, rows_v, sem):
        wid = lax.axis_index("s") * NUM_CORES + lax.axis_index("c")
        base = wid * b_per_w
        pltpu.sync_copy(idx_hbm.at[wid], idx_v)

        def body(i, carry):
            pltpu.async_copy(table_hbm.at[idx_v.at[i]], rows_v, sem).wait()
            pltpu.sync_copy(rows_v, out_hbm.at[pl.ds(base + i * chunk, chunk)])
            return carry

        lax.fori_loop(0, n_chunks, body, 0)

    return gather_kernel(idx3, table)


def _tc_project(low2, proj_w2, b, l):
    n2 = low2.shape[0]
    bm = 1024
    assert b % bm == 0

    def mm_kernel(x_ref, w_ref, o_ref):
        y = jnp.dot(x_ref[...], w_ref[...],
                    preferred_element_type=jnp.float32)
        o_ref[0] = jnp.concatenate([y[:, :D_MODEL], y[:, D_MODEL:]], axis=0)

    grid = (l, b // bm)
    return pl.pallas_call(
        mm_kernel,
        grid=grid,
        in_specs=[
            pl.BlockSpec((bm // 2, 2 * RANK),
                         lambda i, j: (i * (b // bm) + j, 0)),
            pl.BlockSpec((2 * RANK, 2 * D_MODEL), lambda i, j: (0, 0)),
        ],
        out_specs=pl.BlockSpec((1, bm, D_MODEL), lambda i, j: (i, j, 0)),
        out_shape=jax.ShapeDtypeStruct((l, b, D_MODEL), jnp.float32),
    )(low2, proj_w2)


def kernel(input_ids, embed_table, proj_w):
    b, l = input_ids.shape
    n = b * l
    bm = 1024
    ids_perm = (input_ids.T.astype(jnp.int32)
                .reshape(l, b // bm, 2, bm // 2)
                .transpose(0, 1, 3, 2))
    ids_lmajor = ids_perm.reshape(NW, n // (NW * CHUNK), CHUNK)
    low_rank = _sc_gather(ids_lmajor, embed_table)
    low2 = low_rank.reshape(n // 2, 2 * RANK)
    w2 = jnp.zeros((2 * RANK, 2 * D_MODEL), proj_w.dtype)
    w2 = w2.at[:RANK, :D_MODEL].set(proj_w).at[RANK:, D_MODEL:].set(proj_w)
    out3 = _tc_project(low2, w2, b, l)
    return out3.transpose(1, 0, 2)


# speedup vs baseline: 2.2226x; 1.0643x over previous
import functools

import jax
import jax.numpy as jnp
from jax import lax
from jax.experimental import pallas as pl
from jax.experimental.pallas import tpu as pltpu
from jax.experimental.pallas import tpu_sc as plsc

RANK = 64
D_MODEL = 768

NUM_CORES = 2
NUM_SUBCORES = 16
NW = NUM_CORES * NUM_SUBCORES

CHUNK = 128


def _sc_gather(idx3, table):
    nw, n_chunks, chunk = idx3.shape
    n = nw * n_chunks * chunk
    b_per_w = n_chunks * chunk

    mesh = plsc.VectorSubcoreMesh(
        core_axis_name="c", subcore_axis_name="s",
        num_cores=NUM_CORES, num_subcores=NUM_SUBCORES)

    @functools.partial(
        pl.kernel,
        out_type=jax.ShapeDtypeStruct((n, RANK), jnp.float32),
        mesh=mesh,
        scratch_types=[
            pltpu.VMEM((n_chunks, chunk), jnp.int32),
            pltpu.VMEM((chunk, RANK), jnp.float32),
            pltpu.SemaphoreType.DMA,
        ],
        compiler_params=pltpu.CompilerParams(use_tc_tiling_on_sc=False),
    )
    def gather_kernel(idx_hbm, table_hbm, out_hbm, idx_v, rows_v, sem):
        wid = lax.axis_index("s") * NUM_CORES + lax.axis_index("c")
        base = wid * b_per_w
        pltpu.sync_copy(idx_hbm.at[wid], idx_v)

        def body(i, carry):
            pltpu.async_copy(table_hbm.at[idx_v.at[i]], rows_v, sem).wait()
            pltpu.sync_copy(rows_v, out_hbm.at[pl.ds(base + i * chunk, chunk)])
            return carry

        lax.fori_loop(0, n_chunks, body, 0)

    return gather_kernel(idx3, table)


def _tc_project(low2, proj_w2, b, l):
    n2 = low2.shape[0]
    bm = 2048
    assert b % bm == 0

    def mm_kernel(x_ref, w_ref, o_ref):
        y = jnp.dot(x_ref[...], w_ref[...],
                    preferred_element_type=jnp.float32)
        o_ref[0] = jnp.concatenate([y[:, :D_MODEL], y[:, D_MODEL:]], axis=0)

    grid = (l, b // bm)
    return pl.pallas_call(
        mm_kernel,
        grid=grid,
        in_specs=[
            pl.BlockSpec((bm // 2, 2 * RANK),
                         lambda i, j: (i * (b // bm) + j, 0)),
            pl.BlockSpec((2 * RANK, 2 * D_MODEL), lambda i, j: (0, 0)),
        ],
        out_specs=pl.BlockSpec((1, bm, D_MODEL), lambda i, j: (i, j, 0)),
        out_shape=jax.ShapeDtypeStruct((l, b, D_MODEL), jnp.float32),
    )(low2, proj_w2)


def kernel(input_ids, embed_table, proj_w):
    b, l = input_ids.shape
    n = b * l
    bm = 2048
    ids_perm = (input_ids.T.astype(jnp.int32)
                .reshape(l, b // bm, 2, bm // 2)
                .transpose(0, 1, 3, 2))
    ids_lmajor = ids_perm.reshape(NW, n // (NW * CHUNK), CHUNK)
    low_rank = _sc_gather(ids_lmajor, embed_table)
    low2 = low_rank.reshape(n // 2, 2 * RANK)
    w2 = jnp.zeros((2 * RANK, 2 * D_MODEL), proj_w.dtype)
    w2 = w2.at[:RANK, :D_MODEL].set(proj_w).at[RANK:, D_MODEL:].set(proj_w)
    out3 = _tc_project(low2, w2, b, l)
    return out3.transpose(1, 0, 2)


# speedup vs baseline: 2.2373x; 1.0066x over previous
import functools

import jax
import jax.numpy as jnp
from jax import lax
from jax.experimental import pallas as pl
from jax.experimental.pallas import tpu as pltpu
from jax.experimental.pallas import tpu_sc as plsc

RANK = 64
D_MODEL = 768

NUM_CORES = 2
NUM_SUBCORES = 16
NW = NUM_CORES * NUM_SUBCORES

CHUNK = 128


def _sc_gather(idx3, table):
    nw, n_chunks, chunk = idx3.shape
    n = nw * n_chunks * chunk
    b_per_w = n_chunks * chunk

    mesh = plsc.VectorSubcoreMesh(
        core_axis_name="c", subcore_axis_name="s",
        num_cores=NUM_CORES, num_subcores=NUM_SUBCORES)

    @functools.partial(
        pl.kernel,
        out_type=jax.ShapeDtypeStruct((n, RANK), jnp.float32),
        mesh=mesh,
        scratch_types=[
            pltpu.VMEM((n_chunks, chunk), jnp.int32),
            pltpu.VMEM((chunk, RANK), jnp.float32),
            pltpu.SemaphoreType.DMA,
        ],
        compiler_params=pltpu.CompilerParams(use_tc_tiling_on_sc=False),
    )
    def gather_kernel(idx_hbm, table_hbm, out_hbm, idx_v, rows_v, sem):
        wid = lax.axis_index("s") * NUM_CORES + lax.axis_index("c")
        base = wid * b_per_w
        pltpu.sync_copy(idx_hbm.at[wid], idx_v)

        def body(i, carry):
            pltpu.async_copy(table_hbm.at[idx_v.at[i]], rows_v, sem).wait()
            pltpu.sync_copy(rows_v, out_hbm.at[pl.ds(base + i * chunk, chunk)])
            return carry

        lax.fori_loop(0, n_chunks, body, 0)

    return gather_kernel(idx3, table)


def _tc_project(low2, proj_w2, b, l):
    n2 = low2.shape[0]
    bm = 4096
    assert b % bm == 0

    def mm_kernel(x_ref, w_ref, o_ref):
        y = jnp.dot(x_ref[...], w_ref[...],
                    preferred_element_type=jnp.float32)
        o_ref[0] = jnp.concatenate([y[:, :D_MODEL], y[:, D_MODEL:]], axis=0)

    grid = (l, b // bm)
    return pl.pallas_call(
        mm_kernel,
        grid=grid,
        in_specs=[
            pl.BlockSpec((bm // 2, 2 * RANK),
                         lambda i, j: (i * (b // bm) + j, 0)),
            pl.BlockSpec((2 * RANK, 2 * D_MODEL), lambda i, j: (0, 0)),
        ],
        out_specs=pl.BlockSpec((1, bm, D_MODEL), lambda i, j: (i, j, 0)),
        out_shape=jax.ShapeDtypeStruct((l, b, D_MODEL), jnp.float32),
    )(low2, proj_w2)


def kernel(input_ids, embed_table, proj_w):
    b, l = input_ids.shape
    n = b * l
    bm = 4096
    ids_perm = (input_ids.T.astype(jnp.int32)
                .reshape(l, b // bm, 2, bm // 2)
                .transpose(0, 1, 3, 2))
    ids_lmajor = ids_perm.reshape(NW, n // (NW * CHUNK), CHUNK)
    low_rank = _sc_gather(ids_lmajor, embed_table)
    low2 = low_rank.reshape(n // 2, 2 * RANK)
    w2 = jnp.zeros((2 * RANK, 2 * D_MODEL), proj_w.dtype)
    w2 = w2.at[:RANK, :D_MODEL].set(proj_w).at[RANK:, D_MODEL:].set(proj_w)
    out3 = _tc_project(low2, w2, b, l)
    return out3.transpose(1, 0, 2)
